# baseline (speedup 1.0000x reference)
import functools
import math

import jax
import jax.numpy as jnp
from jax import lax
from jax.experimental import pallas as pl
from jax.experimental.pallas import tpu as pltpu
from jax.experimental.pallas import tpu_sc as plsc

N = 10000
E = 320000
D = 128

NC = 2
NS = 16
NW = NC * NS
C = 128
N_PAD = 10240
RPT = N_PAD // NS
DEG_W = 16

CH_TOT = math.ceil(E / (C * NS))
CH0 = 100
CH1 = 57
CHMAX = max(CH0, CH1)
CPAD = (CHMAX + 8 + 7) // 8 * 8
CHD = 80
TROWS = NW * CHD
E_FLAT = TROWS * C
assert NS * CH0 + NS * CH1 >= math.ceil(E / C)
assert (NS * CH0 + (NS - 1) * CH1) // 8 * 8 + CPAD <= TROWS

_MESH = plsc.VectorSubcoreMesh(core_axis_name="c", subcore_axis_name="s")


@functools.partial(
    pl.kernel,
    out_type=jax.ShapeDtypeStruct((NC, N_PAD, DEG_W), jnp.float32),
    mesh=_MESH,
    scratch_types=[
        pltpu.VMEM((CHD, C), jnp.int32),
        pltpu.VMEM((C, DEG_W), jnp.float32),
        pltpu.VMEM_SHARED((N_PAD, DEG_W), jnp.float32),
    ],
)
def _deg_kernel(coli_hbm, zeros_hbm, out_hbm, col_v, ones_v, dacc_sh):
    cid = lax.axis_index("c")
    sid = lax.axis_index("s")
    wid = sid * NC + cid

    e0 = jnp.where(lax.iota(jnp.int32, 16) == 0,
                   jnp.full((16,), 1.0, jnp.float32),
                   jnp.full((16,), 0.0, jnp.float32))

    def _fill(i, carry):
        ones_v[i] = e0
        return carry
    lax.fori_loop(0, C, _fill, 0)

    pltpu.sync_copy(zeros_hbm.at[pl.ds(sid * RPT, RPT)],
                    dacc_sh.at[pl.ds(sid * RPT, RPT)])
    pltpu.sync_copy(coli_hbm.at[wid], col_v)
    plsc.subcore_barrier()

    def _body(j, carry):
        pltpu.sync_copy(ones_v, dacc_sh.at[col_v.at[j]], add=True)
        return carry
    lax.fori_loop(0, CHD, _body, 0)

    plsc.subcore_barrier()
    pltpu.sync_copy(dacc_sh.at[pl.ds(sid * RPT, RPT)],
                    out_hbm.at[cid, pl.ds(sid * RPT, RPT)])


@functools.partial(
    pl.kernel,
    out_type=jax.ShapeDtypeStruct((NC, N_PAD, D), jnp.float32),
    mesh=_MESH,
    scratch_types=[
        pltpu.VMEM((CPAD, C), jnp.int32),
        pltpu.VMEM((CPAD, C), jnp.int32),
        pltpu.VMEM((C, D), jnp.float32),
        pltpu.VMEM_SHARED((N_PAD, D), jnp.float32),
        pltpu.SemaphoreType.DMA,
    ],
)
def _agg_kernel(g_hbm, rowi_hbm, coli_hbm, zeros_hbm, out_hbm,
                row_v, col_v, rows_v, acc_sh, gsem):
    cid = lax.axis_index("c")
    sid = lax.axis_index("s")
    nch = lax.select(cid == 0, jnp.int32(CH0), jnp.int32(CH1))
    rs = lax.select(cid == 0, sid * CH0, NS * CH0 + sid * CH1)
    rs_al = pl.multiple_of((rs // 8) * 8, 8)
    off = rs - rs_al

    pltpu.sync_copy(zeros_hbm.at[pl.ds(sid * RPT, RPT)],
                    acc_sh.at[pl.ds(sid * RPT, RPT)])
    pltpu.sync_copy(rowi_hbm.at[pl.ds(rs_al, CPAD)], row_v)
    pltpu.sync_copy(coli_hbm.at[pl.ds(rs_al, CPAD)], col_v)
    plsc.subcore_barrier()

    def _body(j, carry):
        pltpu.async_copy(g_hbm.at[row_v.at[j + off]], rows_v, gsem).wait()
        pltpu.sync_copy(rows_v, acc_sh.at[col_v.at[j + off]], add=True)
        return carry
    lax.fori_loop(0, nch, _body, 0)

    plsc.subcore_barrier()
    pltpu.sync_copy(acc_sh.at[pl.ds(sid * RPT, RPT)],
                    out_hbm.at[cid, pl.ds(sid * RPT, RPT)])


BM = 2000


def _mm_body(x_ref, w_ref, d0_ref, d1_ref, o_ref):
    deg = d0_ref[0, :, 0:1] + d1_ref[0, :, 0:1] + 1.0
    dinv = lax.rsqrt(deg)
    o_ref[...] = jnp.dot(x_ref[...], w_ref[...],
                         preferred_element_type=jnp.float32) * dinv


def _final_body(a0_ref, a1_ref, g_ref, d0_ref, d1_ref, b_ref, o_ref):
    deg = d0_ref[0, :, 0:1] + d1_ref[0, :, 0:1] + 1.0
    dinv = lax.rsqrt(deg)
    s = dinv * (a0_ref[0] + a1_ref[0] + g_ref[...]) + b_ref[...]
    o_ref[...] = jnp.maximum(s, 0.0)


def kernel(x, edge_index, W, b):
    ei = edge_index.astype(jnp.int32)
    row = ei[0]
    col = ei[1]
    npad = E_FLAT - E
    pad_iota = jnp.arange(npad, dtype=jnp.int32)
    dummy_cols = N + pad_iota % (N_PAD - N)
    dummy_rows = (pad_iota * 37) % N
    rowp = jnp.concatenate([row, dummy_rows]).reshape(TROWS, C)
    colp = jnp.concatenate([col, dummy_cols]).reshape(TROWS, C)

    zeros_deg = jnp.zeros((N_PAD, DEG_W), jnp.float32)
    zeros_acc = jnp.zeros((N_PAD, D), jnp.float32)

    degp = _deg_kernel(colp.reshape(NW, CHD, C), zeros_deg)

    g = pl.pallas_call(
        _mm_body,
        grid=(N // BM,),
        in_specs=[
            pl.BlockSpec((BM, D), lambda i: (i, 0)),
            pl.BlockSpec((D, D), lambda i: (0, 0)),
            pl.BlockSpec((1, BM, DEG_W), lambda i: (0, i, 0)),
            pl.BlockSpec((1, BM, DEG_W), lambda i: (1, i, 0)),
        ],
        out_specs=pl.BlockSpec((BM, D), lambda i: (i, 0)),
        out_shape=jax.ShapeDtypeStruct((N, D), jnp.float32),
    )(x, W, degp, degp)

    acc = _agg_kernel(g, rowp, colp, zeros_acc)

    out = pl.pallas_call(
        _final_body,
        grid=(N // BM,),
        in_specs=[
            pl.BlockSpec((1, BM, D), lambda i: (0, i, 0)),
            pl.BlockSpec((1, BM, D), lambda i: (1, i, 0)),
            pl.BlockSpec((BM, D), lambda i: (i, 0)),
            pl.BlockSpec((1, BM, DEG_W), lambda i: (0, i, 0)),
            pl.BlockSpec((1, BM, DEG_W), lambda i: (1, i, 0)),
            pl.BlockSpec((1, D), lambda i: (0, 0)),
        ],
        out_specs=pl.BlockSpec((BM, D), lambda i: (i, 0)),
        out_shape=jax.ShapeDtypeStruct((N, D), jnp.float32),
    )(acc, acc, g, degp, degp, b.reshape(1, D))

    return out


# speedup vs baseline: 1.1618x; 1.1618x over previous
import functools
import math

import jax
import jax.numpy as jnp
from jax import lax
from jax.experimental import pallas as pl
from jax.experimental.pallas import tpu as pltpu
from jax.experimental.pallas import tpu_sc as plsc

N = 10000
E = 320000
D = 128

NC = 2
NS = 16
NW = NC * NS
C = 128
N_PAD = 10240
RPT = N_PAD // NS
DEG_W = 16

CH_TOT = math.ceil(E / (C * NS))
CH0 = 79
CH1 = 78
CHMAX = max(CH0, CH1)
CPAD = (CHMAX + 8 + 7) // 8 * 8
CHD = 80
TROWS = NW * CHD
E_FLAT = TROWS * C
assert NS * CH0 + NS * CH1 >= math.ceil(E / C)
assert (NS * CH0 + (NS - 1) * CH1) // 8 * 8 + CPAD <= TROWS

_MESH = plsc.VectorSubcoreMesh(core_axis_name="c", subcore_axis_name="s")


@functools.partial(
    pl.kernel,
    out_type=jax.ShapeDtypeStruct((NC, N_PAD, DEG_W), jnp.float32),
    mesh=_MESH,
    scratch_types=[
        pltpu.VMEM((CHD, C), jnp.int32),
        pltpu.VMEM((C, DEG_W), jnp.float32),
        pltpu.VMEM_SHARED((N_PAD, DEG_W), jnp.float32),
    ],
)
def _deg_kernel(coli_hbm, zeros_hbm, out_hbm, col_v, ones_v, dacc_sh):
    cid = lax.axis_index("c")
    sid = lax.axis_index("s")
    wid = sid * NC + cid

    e0 = jnp.where(lax.iota(jnp.int32, 16) == 0,
                   jnp.full((16,), 1.0, jnp.float32),
                   jnp.full((16,), 0.0, jnp.float32))

    def _fill(i, carry):
        ones_v[i] = e0
        return carry
    lax.fori_loop(0, C, _fill, 0)

    pltpu.sync_copy(zeros_hbm.at[pl.ds(sid * RPT, RPT)],
                    dacc_sh.at[pl.ds(sid * RPT, RPT)])
    pltpu.sync_copy(coli_hbm.at[wid], col_v)
    plsc.subcore_barrier()

    def _body(j, carry):
        pltpu.sync_copy(ones_v, dacc_sh.at[col_v.at[j]], add=True)
        return carry
    lax.fori_loop(0, CHD, _body, 0)

    plsc.subcore_barrier()
    pltpu.sync_copy(dacc_sh.at[pl.ds(sid * RPT, RPT)],
                    out_hbm.at[cid, pl.ds(sid * RPT, RPT)])


@functools.partial(
    pl.kernel,
    out_type=jax.ShapeDtypeStruct((NC, N_PAD, D), jnp.float32),
    mesh=_MESH,
    scratch_types=[
        pltpu.VMEM((CPAD, C), jnp.int32),
        pltpu.VMEM((CPAD, C), jnp.int32),
        pltpu.VMEM((C, D), jnp.float32),
        pltpu.VMEM_SHARED((N_PAD, D), jnp.float32),
        pltpu.SemaphoreType.DMA,
    ],
)
def _agg_kernel(g_hbm, rowi_hbm, coli_hbm, zeros_hbm, out_hbm,
                row_v, col_v, rows_v, acc_sh, gsem):
    cid = lax.axis_index("c")
    sid = lax.axis_index("s")
    nch = lax.select(cid == 0, jnp.int32(CH0), jnp.int32(CH1))
    rs = lax.select(cid == 0, sid * CH0, NS * CH0 + sid * CH1)
    rs_al = pl.multiple_of((rs // 8) * 8, 8)
    off = rs - rs_al

    pltpu.sync_copy(zeros_hbm.at[pl.ds(sid * RPT, RPT)],
                    acc_sh.at[pl.ds(sid * RPT, RPT)])
    pltpu.sync_copy(rowi_hbm.at[pl.ds(rs_al, CPAD)], row_v)
    pltpu.sync_copy(coli_hbm.at[pl.ds(rs_al, CPAD)], col_v)
    plsc.subcore_barrier()

    def _body(j, carry):
        pltpu.async_copy(g_hbm.at[row_v.at[j + off]], rows_v, gsem).wait()
        pltpu.sync_copy(rows_v, acc_sh.at[col_v.at[j + off]], add=True)
        return carry
    lax.fori_loop(0, nch, _body, 0)

    plsc.subcore_barrier()
    pltpu.sync_copy(acc_sh.at[pl.ds(sid * RPT, RPT)],
                    out_hbm.at[cid, pl.ds(sid * RPT, RPT)])


BM = 2000


def _mm_body(x_ref, w_ref, d0_ref, d1_ref, o_ref):
    deg = d0_ref[0, :, 0:1] + d1_ref[0, :, 0:1] + 1.0
    dinv = lax.rsqrt(deg)
    o_ref[...] = jnp.dot(x_ref[...], w_ref[...],
                         preferred_element_type=jnp.float32) * dinv


def _final_body(a0_ref, a1_ref, g_ref, d0_ref, d1_ref, b_ref, o_ref):
    deg = d0_ref[0, :, 0:1] + d1_ref[0, :, 0:1] + 1.0
    dinv = lax.rsqrt(deg)
    s = dinv * (a0_ref[0] + a1_ref[0] + g_ref[...]) + b_ref[...]
    o_ref[...] = jnp.maximum(s, 0.0)


def kernel(x, edge_index, W, b):
    ei = edge_index.astype(jnp.int32)
    row = ei[0]
    col = ei[1]
    npad = E_FLAT - E
    pad_iota = jnp.arange(npad, dtype=jnp.int32)
    dummy_cols = N + pad_iota % (N_PAD - N)
    dummy_rows = (pad_iota * 37) % N
    rowp = jnp.concatenate([row, dummy_rows]).reshape(TROWS, C)
    colp = jnp.concatenate([col, dummy_cols]).reshape(TROWS, C)

    zeros_deg = jnp.zeros((N_PAD, DEG_W), jnp.float32)
    zeros_acc = jnp.zeros((N_PAD, D), jnp.float32)

    degp = _deg_kernel(colp.reshape(NW, CHD, C), zeros_deg)

    g = pl.pallas_call(
        _mm_body,
        grid=(N // BM,),
        in_specs=[
            pl.BlockSpec((BM, D), lambda i: (i, 0)),
            pl.BlockSpec((D, D), lambda i: (0, 0)),
            pl.BlockSpec((1, BM, DEG_W), lambda i: (0, i, 0)),
            pl.BlockSpec((1, BM, DEG_W), lambda i: (1, i, 0)),
        ],
        out_specs=pl.BlockSpec((BM, D), lambda i: (i, 0)),
        out_shape=jax.ShapeDtypeStruct((N, D), jnp.float32),
    )(x, W, degp, degp)

    acc = _agg_kernel(g, rowp, colp, zeros_acc)

    out = pl.pallas_call(
        _final_body,
        grid=(N // BM,),
        in_specs=[
            pl.BlockSpec((1, BM, D), lambda i: (0, i, 0)),
            pl.BlockSpec((1, BM, D), lambda i: (1, i, 0)),
            pl.BlockSpec((BM, D), lambda i: (i, 0)),
            pl.BlockSpec((1, BM, DEG_W), lambda i: (0, i, 0)),
            pl.BlockSpec((1, BM, DEG_W), lambda i: (1, i, 0)),
            pl.BlockSpec((1, D), lambda i: (0, 0)),
        ],
        out_specs=pl.BlockSpec((BM, D), lambda i: (i, 0)),
        out_shape=jax.ShapeDtypeStruct((N, D), jnp.float32),
    )(acc, acc, g, degp, degp, b.reshape(1, D))

    return out


# speedup vs baseline: 1.1634x; 1.0014x over previous
import functools
import math

import jax
import jax.numpy as jnp
from jax import lax
from jax.experimental import pallas as pl
from jax.experimental.pallas import tpu as pltpu
from jax.experimental.pallas import tpu_sc as plsc

N = 10000
E = 320000
D = 128

NC = 2
NS = 16
NW = NC * NS
C = 128
N_PAD = 10240
RPT = N_PAD // NS
DEG_W = 16

CH0 = 79
CH1 = 78
CHMAX = max(CH0, CH1)
CPAD = (CHMAX + 8 + 7) // 8 * 8
CHD = 80
TROWS = NW * CHD
E_FLAT = TROWS * C
assert NS * CH0 + NS * CH1 >= math.ceil(E / C)
assert (NS * CH0 + (NS - 1) * CH1) // 8 * 8 + CPAD <= TROWS

_MESH = plsc.VectorSubcoreMesh(core_axis_name="c", subcore_axis_name="s")


@functools.partial(
    pl.kernel,
    out_type=jax.ShapeDtypeStruct((NC, N_PAD, DEG_W), jnp.float32),
    mesh=_MESH,
    scratch_types=[
        pltpu.VMEM((CHD, C), jnp.int32),
        pltpu.VMEM((C, DEG_W), jnp.float32),
        pltpu.VMEM_SHARED((N_PAD, DEG_W), jnp.float32),
    ],
)
def _deg_kernel(coli_hbm, zeros_hbm, out_hbm, col_v, ones_v, dacc_sh):
    cid = lax.axis_index("c")
    sid = lax.axis_index("s")
    wid = sid * NC + cid

    e0 = jnp.where(lax.iota(jnp.int32, 16) == 0,
                   jnp.full((16,), 1.0, jnp.float32),
                   jnp.full((16,), 0.0, jnp.float32))

    def _fill(i, carry):
        ones_v[i] = e0
        return carry
    lax.fori_loop(0, C, _fill, 0)

    pltpu.sync_copy(zeros_hbm.at[pl.ds(sid * RPT, RPT)],
                    dacc_sh.at[pl.ds(sid * RPT, RPT)])
    pltpu.sync_copy(coli_hbm.at[wid], col_v)
    plsc.subcore_barrier()

    def _body(j, carry):
        pltpu.sync_copy(ones_v, dacc_sh.at[col_v.at[j]], add=True)
        return carry
    lax.fori_loop(0, CHD, _body, 0)

    plsc.subcore_barrier()
    pltpu.sync_copy(dacc_sh.at[pl.ds(sid * RPT, RPT)],
                    out_hbm.at[cid, pl.ds(sid * RPT, RPT)])


@functools.partial(
    pl.kernel,
    out_type=jax.ShapeDtypeStruct((NC, N_PAD, D), jnp.float32),
    mesh=_MESH,
    scratch_types=[
        pltpu.VMEM((CPAD, C), jnp.int32),
        pltpu.VMEM((CPAD, C), jnp.int32),
        pltpu.VMEM((C, D), jnp.float32),
        pltpu.VMEM_SHARED((N_PAD, D), jnp.float32),
        pltpu.SemaphoreType.DMA,
    ],
)
def _agg_kernel(g_hbm, rowi_hbm, coli_hbm, zeros_hbm, out_hbm,
                row_v, col_v, rows_v, acc_sh, gsem):
    cid = lax.axis_index("c")
    sid = lax.axis_index("s")
    nch = lax.select(cid == 0, jnp.int32(CH0), jnp.int32(CH1))
    rs = lax.select(cid == 0, sid * CH0, NS * CH0 + sid * CH1)
    rs_al = pl.multiple_of((rs // 8) * 8, 8)
    off = rs - rs_al

    pltpu.sync_copy(zeros_hbm.at[pl.ds(sid * RPT, RPT)],
                    acc_sh.at[pl.ds(sid * RPT, RPT)])
    pltpu.sync_copy(rowi_hbm.at[pl.ds(rs_al, CPAD)], row_v)
    pltpu.sync_copy(coli_hbm.at[pl.ds(rs_al, CPAD)], col_v)
    plsc.subcore_barrier()

    def _body(j, carry):
        pltpu.async_copy(g_hbm.at[row_v.at[j + off]], rows_v, gsem).wait()
        pltpu.sync_copy(rows_v, acc_sh.at[col_v.at[j + off]], add=True)
        return carry
    lax.fori_loop(0, nch, _body, 0)

    plsc.subcore_barrier()
    pltpu.sync_copy(acc_sh.at[pl.ds(sid * RPT, RPT)],
                    out_hbm.at[cid, pl.ds(sid * RPT, RPT)])


BM = 2000


def _mm_body(x_ref, w_ref, d0_ref, d1_ref, o_ref):
    deg = d0_ref[0, :, 0:1] + d1_ref[0, :, 0:1] + 1.0
    dinv = lax.rsqrt(deg)
    o_ref[...] = jnp.dot(x_ref[...], w_ref[...],
                         preferred_element_type=jnp.float32) * dinv


def _final_body(a0_ref, a1_ref, g_ref, d0_ref, d1_ref, b_ref, o_ref):
    deg = d0_ref[0, :, 0:1] + d1_ref[0, :, 0:1] + 1.0
    dinv = lax.rsqrt(deg)
    s = dinv * (a0_ref[0] + a1_ref[0] + g_ref[...]) + b_ref[...]
    o_ref[...] = jnp.maximum(s, 0.0)


def kernel(x, edge_index, W, b):
    ei = edge_index.astype(jnp.int32)
    row = ei[0]
    col = ei[1]
    npad = E_FLAT - E
    pad_iota = jnp.arange(npad, dtype=jnp.int32)
    dummy_cols = N + pad_iota % (N_PAD - N)
    dummy_rows = (pad_iota * 37) % N
    rowp = jnp.concatenate([row, dummy_rows]).reshape(TROWS, C)
    colp = jnp.concatenate([col, dummy_cols]).reshape(TROWS, C)

    zeros_deg = jnp.zeros((N_PAD, DEG_W), jnp.float32)
    zeros_acc = jnp.zeros((N_PAD, D), jnp.float32)

    degp = _deg_kernel(colp.reshape(NW, CHD, C), zeros_deg)

    g = pl.pallas_call(
        _mm_body,
        grid=(N // BM,),
        in_specs=[
            pl.BlockSpec((BM, D), lambda i: (i, 0)),
            pl.BlockSpec((D, D), lambda i: (0, 0)),
            pl.BlockSpec((1, BM, DEG_W), lambda i: (0, i, 0)),
            pl.BlockSpec((1, BM, DEG_W), lambda i: (1, i, 0)),
        ],
        out_specs=pl.BlockSpec((BM, D), lambda i: (i, 0)),
        out_shape=jax.ShapeDtypeStruct((N, D), jnp.float32),
    )(x, W, degp, degp)

    acc = _agg_kernel(g, rowp, colp, zeros_acc)

    out = pl.pallas_call(
        _final_body,
        grid=(N // BM,),
        in_specs=[
            pl.BlockSpec((1, BM, D), lambda i: (0, i, 0)),
            pl.BlockSpec((1, BM, D), lambda i: (1, i, 0)),
            pl.BlockSpec((BM, D), lambda i: (i, 0)),
            pl.BlockSpec((1, BM, DEG_W), lambda i: (0, i, 0)),
            pl.BlockSpec((1, BM, DEG_W), lambda i: (1, i, 0)),
            pl.BlockSpec((1, D), lambda i: (0, 0)),
        ],
        out_specs=pl.BlockSpec((BM, D), lambda i: (i, 0)),
        out_shape=jax.ShapeDtypeStruct((N, D), jnp.float32),
    )(acc, acc, g, degp, degp, b.reshape(1, D))

    return out
